# baseline (speedup 1.0000x reference)
import functools

import jax
import jax.numpy as jnp
from jax import lax
from jax.experimental import pallas as pl
from jax.experimental.pallas import tpu as pltpu
from jax.experimental.pallas import tpu_sc as plsc

_NC = 2
_NS = 16
_NW = _NC * _NS

_D = 128
_K = 10


def _build_sc_gather(b_pad: int, n_nodes: int, chunk: int):
    b_per_w = b_pad // _NW
    assert b_per_w % (2 * chunk) == 0
    n_chunks = b_per_w // chunk

    mesh = plsc.VectorSubcoreMesh(core_axis_name="c", subcore_axis_name="s")

    @functools.partial(
        pl.kernel,
        mesh=mesh,
        out_type=(
            jax.ShapeDtypeStruct((b_pad, _D), jnp.float32),
            jax.ShapeDtypeStruct((b_pad, _D), jnp.float32),
        ),
    scratch_types=[
            pltpu.VMEM((chunk,), jnp.int32),
            pltpu.VMEM((chunk,), jnp.int32),
            pltpu.VMEM((chunk * _K,), jnp.int32),
            pltpu.VMEM((chunk * _K,), jnp.int32),
            pltpu.VMEM((chunk, _D), jnp.float32),
            pltpu.VMEM((chunk, _D), jnp.float32),
            pltpu.VMEM((chunk * _K, _D), jnp.float32),
            pltpu.VMEM((chunk * _K, _D), jnp.float32),
            pltpu.VMEM((chunk, _D), jnp.float32),
            pltpu.SemaphoreType.DMA,
            pltpu.SemaphoreType.DMA,
            pltpu.SemaphoreType.DMA,
            pltpu.SemaphoreType.DMA,
            pltpu.SemaphoreType.DMA,
            pltpu.SemaphoreType.DMA,
        ],
    )
    def sc_gather(nodes_hbm, neigh_hbm, table_hbm, self_out, nsum_out,
                  sidx0, sidx1, nidx0, nidx1, srows0, srows1, nrows0, nrows1,
                  nsum_v, isem0, isem1, ssem0, ssem1, nsem0, nsem1):
        wid = lax.axis_index("s") * _NC + lax.axis_index("c")
        base = wid * b_per_w
        sidx = (sidx0, sidx1)
        nidx = (nidx0, nidx1)
        srows = (srows0, srows1)
        nrows = (nrows0, nrows1)
        isem = (isem0, isem1)
        ssem = (ssem0, ssem1)
        nsem = (nsem0, nsem1)

        def idx_load(g, p):
            off = base + g * chunk
            pltpu.async_copy(nodes_hbm.at[pl.ds(off, chunk)], sidx[p], isem[p])
            pltpu.async_copy(neigh_hbm.at[pl.ds(off * _K, chunk * _K)],
                             nidx[p], isem[p])

        def idx_wait(g, p):
            pltpu.make_async_copy(nodes_hbm.at[pl.ds(0, chunk)], sidx[p],
                                  isem[p]).wait()
            pltpu.make_async_copy(neigh_hbm.at[pl.ds(0, chunk * _K)], nidx[p],
                                  isem[p]).wait()

        def gather(g, p):
            idx_wait(g, p)
            pltpu.async_copy(table_hbm.at[sidx[p]], srows[p], ssem[p])
            pltpu.async_copy(table_hbm.at[nidx[p]], nrows[p], nsem[p])

        def gather_wait(p):
            pltpu.make_async_copy(table_hbm.at[sidx[p]], srows[p], ssem[p]).wait()
            pltpu.make_async_copy(table_hbm.at[nidx[p]], nrows[p], nsem[p]).wait()

        def compute(g, p):
            off = base + g * chunk
            pltpu.sync_copy(srows[p], self_out.at[pl.ds(off, chunk)])
            nr = nrows[p]

            @pl.loop(0, chunk)
            def _node_loop(i):
                r0 = i * _K
                for c in range(_D // 16):
                    sl = pl.ds(c * 16, 16)
                    acc = nr[r0, sl]
                    for j in range(1, _K):
                        acc = acc + nr[r0 + j, sl]
                    nsum_v[i, sl] = acc

            pltpu.sync_copy(nsum_v, nsum_out.at[pl.ds(off, chunk)])

        idx_load(0, 0)
        idx_load(1, 1)
        gather(0, 0)

        @pl.loop(0, n_chunks, step=2)
        def _chunk_loop(g):
            gather(g + 1, 1)
            gather_wait(0)

            @pl.when(g + 2 < n_chunks)
            def _():
                idx_load(g + 2, 0)

            compute(g, 0)

            @pl.when(g + 2 < n_chunks)
            def _():
                gather(g + 2, 0)

            gather_wait(1)

            @pl.when(g + 3 < n_chunks)
            def _():
                idx_load(g + 3, 1)

            compute(g + 1, 1)

    return sc_gather


def _tc_matmul_body(s_ref, n_ref, w_ref, o_ref):
    s = s_ref[...]
    n = n_ref[...] * (1.0 / _K)
    acc = jnp.dot(s, w_ref[0:_D, :], preferred_element_type=jnp.float32)
    acc = acc + jnp.dot(n, w_ref[_D:2 * _D, :], preferred_element_type=jnp.float32)
    o_ref[...] = jnp.maximum(acc, 0.0)


def _tc_matmul(self_rows, nsum, w, bm: int):
    b_pad = self_rows.shape[0]
    grid = (b_pad // bm,)
    return pl.pallas_call(
        _tc_matmul_body,
        grid=grid,
        in_specs=[
            pl.BlockSpec((bm, _D), lambda i: (i, 0)),
            pl.BlockSpec((bm, _D), lambda i: (i, 0)),
            pl.BlockSpec((2 * _D, _D), lambda i: (0, 0)),
        ],
        out_specs=pl.BlockSpec((bm, _D), lambda i: (i, 0)),
        out_shape=jax.ShapeDtypeStruct((b_pad, _D), jnp.float32),
    )(self_rows, nsum, w)


def kernel(nodes, neigh_idx, feat_table, W):
    b = nodes.shape[0]
    n_nodes = feat_table.shape[0]

    chunk = 32
    unit = _NW * chunk * 2
    b_pad = ((b + unit - 1) // unit) * unit
    pad = b_pad - b

    nodes_p = jnp.pad(nodes, (0, pad))
    neigh_flat = jnp.pad(neigh_idx.reshape(-1), (0, pad * _K))

    sc = _build_sc_gather(b_pad, n_nodes, chunk)
    self_rows, nsum = sc(nodes_p, neigh_flat, feat_table)

    out = _tc_matmul(self_rows, nsum, W, bm=1024)
    return out[:b]
